# baseline (speedup 1.0000x reference)
import functools
import math

import jax
import jax.numpy as jnp
from jax import lax
from jax.experimental import pallas as pl
from jax.experimental.pallas import tpu as pltpu
from jax.experimental.pallas import tpu_sc as plsc

_NUM_CORES = 2
_NUM_SUBCORES = 16
_NUM_WORKERS = _NUM_CORES * _NUM_SUBCORES
_LANES = 16
_CHUNK = 128


@jax.jit
def _sc_lpe(xf, mk, idx, pe_table):
    n, d = xf.shape
    v = pe_table.shape[0]
    pad = v - 1
    scale = math.sqrt(d)
    per_w = n // _NUM_WORKERS
    n_chunks = per_w // _CHUNK
    mesh = plsc.VectorSubcoreMesh(core_axis_name="c", subcore_axis_name="s")

    @functools.partial(
        pl.kernel,
        mesh=mesh,
        out_type=jax.ShapeDtypeStruct((n, d), jnp.float32),
        scratch_types=[
            pltpu.VMEM((_CHUNK,), jnp.int32),
            pltpu.VMEM((_CHUNK,), jnp.int32),
            pltpu.VMEM((_CHUNK, d), jnp.float32),
            pltpu.VMEM((_CHUNK, d), jnp.float32),
            pltpu.SemaphoreType.DMA,
        ],
    )
    def k(x_hbm, mk_hbm, idx_hbm, tab_hbm, out_hbm, idx_v, mk_v, x_v, rows_v, sem):
        wid = lax.axis_index("s") * _NUM_CORES + lax.axis_index("c")
        base_w = wid * per_w

        def chunk_body(c, carry):
            base = base_w + c * _CHUNK
            pltpu.sync_copy(idx_hbm.at[pl.ds(base, _CHUNK)], idx_v)
            pltpu.sync_copy(mk_hbm.at[pl.ds(base, _CHUNK)], mk_v)

            def idx_body(i, carry2):
                off = i * _LANES
                ii = idx_v[pl.ds(off, _LANES)]
                mm = mk_v[pl.ds(off, _LANES)]
                idx_v[pl.ds(off, _LANES)] = jnp.where(
                    mm != 0, pad, jnp.minimum(ii, pad)
                )
                return carry2

            lax.fori_loop(0, _CHUNK // _LANES, idx_body, 0)

            cp_rows = pltpu.async_copy(tab_hbm.at[idx_v], rows_v, sem)
            cp_x = pltpu.async_copy(x_hbm.at[pl.ds(base, _CHUNK)], x_v, sem)
            cp_rows.wait()
            cp_x.wait()

            def fma_body(i, carry2):
                for j in range(d // _LANES):
                    sl = pl.ds(j * _LANES, _LANES)
                    rows_v[i, sl] = scale * x_v[i, sl] + rows_v[i, sl]
                return carry2

            lax.fori_loop(0, _CHUNK, fma_body, 0)
            pltpu.sync_copy(rows_v, out_hbm.at[pl.ds(base, _CHUNK)])
            return carry

        lax.fori_loop(0, n_chunks, chunk_body, 0)

    return k(xf, mk, idx, pe_table)


def kernel(x, mask, indices, pe_table):
    b, s, d = x.shape
    n = b * s
    xf = x.reshape(n, d)
    mk = mask.reshape(n).astype(jnp.int32)
    idx = indices.reshape(n).astype(jnp.int32)
    out = _sc_lpe(xf, mk, idx, pe_table)
    return out.reshape(b, s, d)


# speedup vs baseline: 1.0001x; 1.0001x over previous
import functools
import math

import jax
import jax.numpy as jnp
from jax import lax
from jax.experimental import pallas as pl
from jax.experimental.pallas import tpu as pltpu
from jax.experimental.pallas import tpu_sc as plsc

_NUM_CORES = 2
_NUM_SUBCORES = 16
_NUM_WORKERS = _NUM_CORES * _NUM_SUBCORES
_LANES = 16
_C = 64
_NBUF = 4


@jax.jit
def _sc_lpe(xf, mk, idx, pe_table):
    n, d = xf.shape
    v = pe_table.shape[0]
    pad = v - 1
    scale = math.sqrt(d)
    per_w = n // _NUM_WORKERS
    n_chunks = per_w // _C
    assert n_chunks % _NBUF == 0
    mk2 = mk.reshape(_NUM_WORKERS, per_w)
    idx2 = idx.reshape(_NUM_WORKERS, per_w)
    mesh = plsc.VectorSubcoreMesh(core_axis_name="c", subcore_axis_name="s")

    @functools.partial(
        pl.kernel,
        mesh=mesh,
        out_type=jax.ShapeDtypeStruct((n, d), jnp.float32),
        scratch_types=[
            pltpu.VMEM((per_w,), jnp.int32),
            pltpu.VMEM((per_w,), jnp.int32),
            *[pltpu.VMEM((_C, d), jnp.float32) for _ in range(2 * _NBUF)],
            *[pltpu.SemaphoreType.DMA for _ in range(2 * _NBUF)],
        ],
    )
    def k(x_hbm, mk_hbm, idx_hbm, tab_hbm, out_hbm, idx_v, mk_v, *bufs):
        rows = bufs[0:_NBUF]
        xs = bufs[_NBUF : 2 * _NBUF]
        sin = bufs[2 * _NBUF : 3 * _NBUF]
        sout = bufs[3 * _NBUF : 4 * _NBUF]
        wid = lax.axis_index("s") * _NUM_CORES + lax.axis_index("c")
        base_w = wid * per_w

        pltpu.sync_copy(idx_hbm.at[wid], idx_v)
        pltpu.sync_copy(mk_hbm.at[wid], mk_v)

        def _pad(i, carry):
            sl = pl.ds(i * _LANES, _LANES)
            idx_v[sl] = jnp.where(mk_v[sl] != 0, pad, jnp.minimum(idx_v[sl], pad))
            return carry

        lax.fori_loop(0, per_w // _LANES, _pad, 0)

        def issue_in(c, b):
            pltpu.async_copy(tab_hbm.at[idx_v.at[pl.ds(c * _C, _C)]], rows[b], sin[b])
            pltpu.async_copy(x_hbm.at[pl.ds(base_w + c * _C, _C)], xs[b], sin[b])

        def wait_in(c, b):
            pltpu.make_async_copy(
                tab_hbm.at[idx_v.at[pl.ds(c * _C, _C)]], rows[b], sin[b]
            ).wait()
            pltpu.make_async_copy(
                x_hbm.at[pl.ds(base_w + c * _C, _C)], xs[b], sin[b]
            ).wait()

        def issue_out(c, b):
            pltpu.async_copy(rows[b], out_hbm.at[pl.ds(base_w + c * _C, _C)], sout[b])

        def wait_out(c, b):
            pltpu.make_async_copy(
                rows[b], out_hbm.at[pl.ds(base_w + c * _C, _C)], sout[b]
            ).wait()

        for b in range(_NBUF - 1):
            issue_in(b, b)

        @pl.loop(0, n_chunks, step=_NBUF)
        def _main(g):
            for b in range(_NBUF):
                c = g + b
                wait_in(c, b)

                def _fma(i, carry):
                    for j in range(d // _LANES):
                        sl = pl.ds(j * _LANES, _LANES)
                        rows[b][i, sl] = scale * xs[b][i, sl] + rows[b][i, sl]
                    return carry

                lax.fori_loop(0, _C, _fma, 0)

                issue_out(c, b)
                nxt = c + _NBUF - 1
                bp = (b + _NBUF - 1) % _NBUF

                @pl.when(nxt < n_chunks)
                def _():
                    @pl.when(c >= 1)
                    def _():
                        wait_out(c - 1, bp)

                    issue_in(nxt, bp)

        for b in range(_NBUF):
            wait_out(n_chunks - _NBUF + b, b)

    return k(xf, mk2, idx2, pe_table)


def kernel(x, mask, indices, pe_table):
    b, s, d = x.shape
    n = b * s
    xf = x.reshape(n, d)
    mk = mask.reshape(n).astype(jnp.int32)
    idx = indices.reshape(n).astype(jnp.int32)
    out = _sc_lpe(xf, mk, idx, pe_table)
    return out.reshape(b, s, d)
